# Initial kernel scaffold; baseline (speedup 1.0000x reference)
import jax
import jax.numpy as jnp
from jax.experimental import pallas as pl


def kernel(batch, seg_labels, weight):
    raise NotImplementedError("write your pallas kernel here")


# speedup vs baseline: 1.5080x; 1.5080x over previous
import functools

import jax
import jax.numpy as jnp
from jax import lax
from jax.experimental import pallas as pl
from jax.experimental.pallas import tpu as pltpu
from jax.experimental.pallas import tpu_sc as plsc

EMBEDDING_DIM = 128
NUM_POS = 8192
PAD_IDX = 1
B, S = 4, 8192
N_TOK = B * S

NC, NS = 2, 16
NW = NC * NS
TOK_PER_W = N_TOK // NW
CHUNK = 128
CHUNKS = TOK_PER_W // CHUNK


def _idx_body(batch_ref, seg_ref, idx_ref):
    b = batch_ref[...]
    seg = seg_ref[...]
    mask = b != PAD_IDX
    m = mask.astype(jnp.int32)
    c = m
    shift = 1
    while shift < S:
        shifted = jnp.concatenate(
            [jnp.zeros((B, shift), jnp.int32), c[:, : S - shift]], axis=1
        )
        c = c + shifted
        shift *= 2
    positions = c * m + PAD_IDX
    idx_ref[...] = jnp.where(mask, positions + NUM_POS * seg, PAD_IDX)


_idx_call = pl.pallas_call(
    _idx_body,
    out_shape=jax.ShapeDtypeStruct((B, S), jnp.int32),
)


_sc_mesh = plsc.VectorSubcoreMesh(core_axis_name="c", subcore_axis_name="s")


@functools.partial(
    pl.kernel,
    mesh=_sc_mesh,
    out_type=jax.ShapeDtypeStruct((N_TOK, EMBEDDING_DIM), jnp.float32),
    scratch_types=[
        pltpu.VMEM((CHUNKS, CHUNK), jnp.int32),
        pltpu.VMEM((CHUNK, EMBEDDING_DIM), jnp.float32),
        pltpu.VMEM((CHUNK, EMBEDDING_DIM), jnp.float32),
        pltpu.SemaphoreType.DMA,
        pltpu.SemaphoreType.DMA,
    ],
)
def _gather_kernel(weight_hbm, idx_hbm, out_hbm, idx_v, rows0, rows1, sem0, sem1):
    wid = lax.axis_index("s") * NC + lax.axis_index("c")
    base = wid * TOK_PER_W
    pltpu.sync_copy(idx_hbm.at[wid], idx_v)
    rows = (rows0, rows1)
    sems = (sem0, sem1)
    pltpu.async_copy(weight_hbm.at[idx_v.at[0]], rows0, sem0)
    for j in range(CHUNKS):
        if j + 1 < CHUNKS:
            pltpu.async_copy(
                weight_hbm.at[idx_v.at[j + 1]], rows[(j + 1) % 2], sems[(j + 1) % 2]
            )
        pltpu.make_async_copy(
            weight_hbm.at[idx_v.at[j]], rows[j % 2], sems[j % 2]
        ).wait()
        pltpu.sync_copy(rows[j % 2], out_hbm.at[pl.ds(base + j * CHUNK, CHUNK)])


def kernel(batch, seg_labels, weight):
    batch = batch.astype(jnp.int32)
    seg_labels = seg_labels.astype(jnp.int32)
    idx = _idx_call(batch, seg_labels)
    idx3 = idx.reshape(NW, CHUNKS, CHUNK)
    out = _gather_kernel(weight, idx3)
    return out.reshape(B, S, EMBEDDING_DIM)


# speedup vs baseline: 1.5958x; 1.0582x over previous
import functools

import jax
import jax.numpy as jnp
from jax import lax
from jax.experimental import pallas as pl
from jax.experimental.pallas import tpu as pltpu
from jax.experimental.pallas import tpu_sc as plsc

EMBEDDING_DIM = 128
NUM_POS = 8192
PAD_IDX = 1
B, S = 4, 8192
N_TOK = B * S

NC, NS = 2, 16
NW = NC * NS
TOK_PER_W = N_TOK // NW
W_PER_ROW = S // TOK_PER_W
CHUNK = 128
CHUNKS = TOK_PER_W // CHUNK
NBUF = 4


def _idx_body(batch_ref, seg_ref, idx_ref):
    b = batch_ref[...]
    seg = seg_ref[...]
    mask = b != PAD_IDX
    m = mask.astype(jnp.int32)
    c = m
    shift = 1
    while shift < S:
        shifted = jnp.concatenate(
            [jnp.zeros((B, shift), jnp.int32), c[:, : S - shift]], axis=1
        )
        c = c + shifted
        shift *= 2
    positions = c * m + PAD_IDX
    idx_ref[...] = jnp.where(mask, positions + NUM_POS * seg, PAD_IDX)


_idx_call = pl.pallas_call(
    _idx_body,
    out_shape=jax.ShapeDtypeStruct((B, S), jnp.int32),
)


_sc_mesh = plsc.VectorSubcoreMesh(core_axis_name="c", subcore_axis_name="s")


@functools.partial(
    pl.kernel,
    mesh=_sc_mesh,
    out_type=jax.ShapeDtypeStruct((N_TOK, EMBEDDING_DIM), jnp.float32),
    scratch_types=[
        pltpu.VMEM((TOK_PER_W,), jnp.int32),
        *[pltpu.VMEM((CHUNK, EMBEDDING_DIM), jnp.float32) for _ in range(NBUF)],
        *[pltpu.SemaphoreType.DMA for _ in range(2 * NBUF)],
    ],
)
def _gather_kernel(weight_hbm, idx_hbm, out_hbm, idx_v, *bufs_and_sems):
    rows = bufs_and_sems[:NBUF]
    gsem = bufs_and_sems[NBUF : 2 * NBUF]
    wsem = bufs_and_sems[2 * NBUF :]
    wid = lax.axis_index("s") * NC + lax.axis_index("c")
    row_b = wid // W_PER_ROW
    col0 = (wid % W_PER_ROW) * TOK_PER_W
    base = wid * TOK_PER_W
    pltpu.sync_copy(idx_hbm.at[row_b, pl.ds(col0, TOK_PER_W)], idx_v)

    def g_start(j):
        pltpu.async_copy(
            weight_hbm.at[idx_v.at[pl.ds(j * CHUNK, CHUNK)]],
            rows[j % NBUF],
            gsem[j % NBUF],
        )

    def g_wait(j):
        pltpu.make_async_copy(
            weight_hbm.at[idx_v.at[pl.ds(j * CHUNK, CHUNK)]],
            rows[j % NBUF],
            gsem[j % NBUF],
        ).wait()

    def w_start(j):
        pltpu.async_copy(
            rows[j % NBUF],
            out_hbm.at[pl.ds(base + j * CHUNK, CHUNK)],
            wsem[j % NBUF],
        )

    def w_wait(j):
        pltpu.make_async_copy(
            rows[j % NBUF],
            out_hbm.at[pl.ds(base + j * CHUNK, CHUNK)],
            wsem[j % NBUF],
        ).wait()

    for j in range(NBUF - 1):
        g_start(j)
    for j in range(CHUNKS):
        g_wait(j)
        w_start(j)
        if j + NBUF - 1 < CHUNKS:
            if j >= 1:
                w_wait(j - 1)
            g_start(j + NBUF - 1)
    for j in range(CHUNKS - NBUF, CHUNKS):
        w_wait(j)


def kernel(batch, seg_labels, weight):
    batch = batch.astype(jnp.int32)
    seg_labels = seg_labels.astype(jnp.int32)
    idx = _idx_call(batch, seg_labels)
    out = _gather_kernel(weight, idx)
    return out.reshape(B, S, EMBEDDING_DIM)


# speedup vs baseline: 1.6125x; 1.0105x over previous
import functools

import jax
import jax.numpy as jnp
from jax import lax
from jax.experimental import pallas as pl
from jax.experimental.pallas import tpu as pltpu
from jax.experimental.pallas import tpu_sc as plsc

EMBEDDING_DIM = 128
NUM_POS = 8192
PAD_IDX = 1
B, S = 4, 8192
N_TOK = B * S

NC, NS = 2, 16
NW = NC * NS
TOK_PER_W = N_TOK // NW
W_PER_ROW = S // TOK_PER_W
CHUNK = 128
CHUNKS = TOK_PER_W // CHUNK
NBUF = 6


def _idx_body(batch_ref, seg_ref, idx_ref):
    b = batch_ref[...]
    seg = seg_ref[...]
    mask = b != PAD_IDX
    m = mask.astype(jnp.int32)
    c = m
    shift = 1
    while shift < S:
        shifted = jnp.concatenate(
            [jnp.zeros((B, shift), jnp.int32), c[:, : S - shift]], axis=1
        )
        c = c + shifted
        shift *= 2
    positions = c * m + PAD_IDX
    idx_ref[...] = jnp.where(mask, positions + NUM_POS * seg, PAD_IDX)


_idx_call = pl.pallas_call(
    _idx_body,
    out_shape=jax.ShapeDtypeStruct((B, S), jnp.int32),
)


_sc_mesh = plsc.VectorSubcoreMesh(core_axis_name="c", subcore_axis_name="s")


@functools.partial(
    pl.kernel,
    mesh=_sc_mesh,
    out_type=jax.ShapeDtypeStruct((N_TOK, EMBEDDING_DIM), jnp.float32),
    scratch_types=[
        pltpu.VMEM((TOK_PER_W,), jnp.int32),
        *[pltpu.VMEM((CHUNK, EMBEDDING_DIM), jnp.float32) for _ in range(NBUF)],
        *[pltpu.SemaphoreType.DMA for _ in range(2 * NBUF)],
    ],
)
def _gather_kernel(weight_hbm, idx_hbm, out_hbm, idx_v, *bufs_and_sems):
    rows = bufs_and_sems[:NBUF]
    gsem = bufs_and_sems[NBUF : 2 * NBUF]
    wsem = bufs_and_sems[2 * NBUF :]
    wid = lax.axis_index("s") * NC + lax.axis_index("c")
    row_b = wid // W_PER_ROW
    col0 = (wid % W_PER_ROW) * TOK_PER_W
    base = wid * TOK_PER_W
    pltpu.sync_copy(idx_hbm.at[row_b, pl.ds(col0, TOK_PER_W)], idx_v)

    def g_start(j):
        pltpu.async_copy(
            weight_hbm.at[idx_v.at[pl.ds(j * CHUNK, CHUNK)]],
            rows[j % NBUF],
            gsem[j % NBUF],
        )

    def g_wait(j):
        pltpu.make_async_copy(
            weight_hbm.at[idx_v.at[pl.ds(j * CHUNK, CHUNK)]],
            rows[j % NBUF],
            gsem[j % NBUF],
        ).wait()

    def w_start(j):
        pltpu.async_copy(
            rows[j % NBUF],
            out_hbm.at[pl.ds(base + j * CHUNK, CHUNK)],
            wsem[j % NBUF],
        )

    def w_wait(j):
        pltpu.make_async_copy(
            rows[j % NBUF],
            out_hbm.at[pl.ds(base + j * CHUNK, CHUNK)],
            wsem[j % NBUF],
        ).wait()

    for j in range(NBUF - 1):
        g_start(j)
    for j in range(CHUNKS):
        g_wait(j)
        w_start(j)
        if j + NBUF - 1 < CHUNKS:
            if j >= 1:
                w_wait(j - 1)
            g_start(j + NBUF - 1)
    for j in range(CHUNKS - NBUF, CHUNKS):
        w_wait(j)


def kernel(batch, seg_labels, weight):
    batch = batch.astype(jnp.int32)
    seg_labels = seg_labels.astype(jnp.int32)
    idx = _idx_call(batch, seg_labels)
    out = _gather_kernel(weight, idx)
    return out.reshape(B, S, EMBEDDING_DIM)


# speedup vs baseline: 1.6386x; 1.0162x over previous
import functools

import jax
import jax.numpy as jnp
from jax import lax
from jax.experimental import pallas as pl
from jax.experimental.pallas import tpu as pltpu
from jax.experimental.pallas import tpu_sc as plsc

EMBEDDING_DIM = 128
NUM_POS = 8192
PAD_IDX = 1
B, S = 4, 8192
N_TOK = B * S

NC, NS = 2, 16
NW = NC * NS
TOK_PER_W = N_TOK // NW
W_PER_ROW = S // TOK_PER_W
CHUNK = 128
CHUNKS = TOK_PER_W // CHUNK
NBUF = 7


def _idx_body(batch_ref, seg_ref, idx_ref):
    b = batch_ref[...]
    seg = seg_ref[...]
    mask = b != PAD_IDX
    m = mask.astype(jnp.int32)
    c = m
    shift = 1
    while shift < S:
        shifted = jnp.concatenate(
            [jnp.zeros((B, shift), jnp.int32), c[:, : S - shift]], axis=1
        )
        c = c + shifted
        shift *= 2
    positions = c * m + PAD_IDX
    idx_ref[...] = jnp.where(mask, positions + NUM_POS * seg, PAD_IDX)


_idx_call = pl.pallas_call(
    _idx_body,
    out_shape=jax.ShapeDtypeStruct((B, S), jnp.int32),
)


_sc_mesh = plsc.VectorSubcoreMesh(core_axis_name="c", subcore_axis_name="s")


@functools.partial(
    pl.kernel,
    mesh=_sc_mesh,
    out_type=jax.ShapeDtypeStruct((N_TOK, EMBEDDING_DIM), jnp.float32),
    scratch_types=[
        pltpu.VMEM((TOK_PER_W,), jnp.int32),
        *[pltpu.VMEM((CHUNK, EMBEDDING_DIM), jnp.float32) for _ in range(NBUF)],
        *[pltpu.SemaphoreType.DMA for _ in range(2 * NBUF)],
    ],
)
def _gather_kernel(weight_hbm, idx_hbm, out_hbm, idx_v, *bufs_and_sems):
    rows = bufs_and_sems[:NBUF]
    gsem = bufs_and_sems[NBUF : 2 * NBUF]
    wsem = bufs_and_sems[2 * NBUF :]
    wid = lax.axis_index("s") * NC + lax.axis_index("c")
    row_b = wid // W_PER_ROW
    col0 = (wid % W_PER_ROW) * TOK_PER_W
    base = wid * TOK_PER_W
    pltpu.sync_copy(idx_hbm.at[row_b, pl.ds(col0, TOK_PER_W)], idx_v)

    def g_start(j):
        pltpu.async_copy(
            weight_hbm.at[idx_v.at[pl.ds(j * CHUNK, CHUNK)]],
            rows[j % NBUF],
            gsem[j % NBUF],
        )

    def g_wait(j):
        pltpu.make_async_copy(
            weight_hbm.at[idx_v.at[pl.ds(j * CHUNK, CHUNK)]],
            rows[j % NBUF],
            gsem[j % NBUF],
        ).wait()

    def w_start(j):
        pltpu.async_copy(
            rows[j % NBUF],
            out_hbm.at[pl.ds(base + j * CHUNK, CHUNK)],
            wsem[j % NBUF],
        )

    def w_wait(j):
        pltpu.make_async_copy(
            rows[j % NBUF],
            out_hbm.at[pl.ds(base + j * CHUNK, CHUNK)],
            wsem[j % NBUF],
        ).wait()

    for j in range(NBUF - 1):
        g_start(j)
    for j in range(CHUNKS):
        g_wait(j)
        w_start(j)
        if j + NBUF - 1 < CHUNKS:
            if j >= 1:
                w_wait(j - 1)
            g_start(j + NBUF - 1)
    for j in range(CHUNKS - NBUF, CHUNKS):
        w_wait(j)


def kernel(batch, seg_labels, weight):
    batch = batch.astype(jnp.int32)
    seg_labels = seg_labels.astype(jnp.int32)
    idx = _idx_call(batch, seg_labels)
    out = _gather_kernel(weight, idx)
    return out.reshape(B, S, EMBEDDING_DIM)
